# baseline (speedup 1.0000x reference)
import functools
import math

import jax
import jax.numpy as jnp
from jax import lax
from jax.experimental import pallas as pl
from jax.experimental.pallas import tpu as pltpu
from jax.experimental.pallas import tpu_sc as plsc

D = 32
V = 1000000
SCALE = math.sqrt(D)

_NC = 2
_NS = 16
_NW = _NC * _NS


def _iota16():
    return lax.iota(jnp.int32, 16)


def _make_repack():
    mesh = plsc.VectorSubcoreMesh(core_axis_name="c", subcore_axis_name="s")

    @functools.partial(
        pl.kernel,
        out_type=jax.ShapeDtypeStruct((V // 4, 128), jnp.float32),
        mesh=mesh,
        scratch_types=[
            pltpu.VMEM((32, 128), jnp.float32),
            pltpu.VMEM((32, 128), jnp.float32),
        ],
        compiler_params=pltpu.CompilerParams(
            use_tc_tiling_on_sc=True, needs_layout_passes=False
        ),
    )
    def repack(tabT_hbm, tail_hbm, packed_hbm, s_v, p_v):
        wid = lax.axis_index("s") * _NC + lax.axis_index("c")
        base = wid * 244 + jnp.minimum(wid, 5)
        cnt = jnp.where(wid < 5, 245, 244)

        it = _iota16()

        def transpose_rows(nrows):
            def row(r, _):
                for g in range(8):
                    rows = it + 16 * (g % 2)
                    cols = jnp.full((16,), 4 * r + g // 2, jnp.int32)
                    val = plsc.load_gather(s_v, [rows, cols])
                    p_v[r, pl.ds(16 * g, 16)] = val * SCALE
                return 0

            lax.fori_loop(0, nrows, row, 0)

        def body(i, _):
            blk = base + i

            @pl.when(blk < 7812)
            def _full():
                v0 = pl.multiple_of(blk * 128, 128)
                pltpu.sync_copy(tabT_hbm.at[:, pl.ds(v0, 128)], s_v)
                transpose_rows(32)
                p0 = pl.multiple_of(blk * 32, 32)
                pltpu.sync_copy(p_v, packed_hbm.at[pl.ds(p0, 32)])

            @pl.when(blk == 7812)
            def _tail():
                pltpu.sync_copy(tail_hbm, p_v.at[pl.ds(0, 16)])
                pltpu.sync_copy(
                    p_v.at[pl.ds(0, 16)], packed_hbm.at[pl.ds(7812 * 32, 16)]
                )

            return 0

        lax.fori_loop(0, cnt, body, 0)

    return repack


def _make_lookup():
    mesh = plsc.VectorSubcoreMesh(core_axis_name="c", subcore_axis_name="s")

    @functools.partial(
        pl.kernel,
        out_type=jax.ShapeDtypeStruct((200, D, 4096), jnp.float32),
        mesh=mesh,
        scratch_types=[
            pltpu.VMEM((8, 128), jnp.int32),
            pltpu.VMEM((128,), jnp.int32),
            pltpu.VMEM((128, 128), jnp.float32),
            pltpu.VMEM((32, 128), jnp.float32),
            pltpu.SemaphoreType.DMA,
        ],
        compiler_params=pltpu.CompilerParams(
            use_tc_tiling_on_sc=True, needs_layout_passes=False
        ),
    )
    def lookup(xT_hbm, packed_hbm, out_hbm, idx_t, jrow_v, g_v, o_v, sem):
        wid = lax.axis_index("s") * _NC + lax.axis_index("c")
        lane0 = wid * 128
        it = _iota16()

        def sblock(sb, _):
            pltpu.sync_copy(
                xT_hbm.at[pl.ds(sb * 8, 8), pl.ds(lane0, 128)], idx_t
            )

            def seq(sl, _):
                koffs = []
                for g in range(8):
                    v = idx_t[sl, pl.ds(16 * g, 16)]
                    jrow_v[pl.ds(16 * g, 16)] = lax.shift_right_logical(v, 2)
                    koffs.append(lax.shift_left(jnp.bitwise_and(v, 3), 5))
                pltpu.async_copy(packed_hbm.at[jrow_v], g_v, sem).wait()
                for dh in range(4):
                    for dl in range(8):
                        d = dh * 8 + dl
                        for g in range(8):
                            rows = it + 16 * g
                            cols = koffs[g] + d
                            o_v[d, pl.ds(16 * g, 16)] = plsc.load_gather(
                                g_v, [rows, cols]
                            )
                pltpu.sync_copy(
                    o_v, out_hbm.at[sb * 8 + sl, :, pl.ds(lane0, 128)]
                )
                return 0

            lax.fori_loop(0, 8, seq, 0)
            return 0

        lax.fori_loop(0, 25, sblock, 0)

    return lookup


def kernel(x, table):
    xT = x.T
    tabT = table.T
    tail = (table[7812 * 128 :, :] * SCALE).reshape(16, 128)
    packed = _make_repack()(tabT, tail)
    outP = _make_lookup()(xT, packed)
    return outP.transpose(2, 0, 1)


# speedup vs baseline: 1.2359x; 1.2359x over previous
import functools
import math

import jax
import jax.numpy as jnp
from jax import lax
from jax.experimental import pallas as pl
from jax.experimental.pallas import tpu as pltpu
from jax.experimental.pallas import tpu_sc as plsc

D = 32
V = 1000000
SCALE = math.sqrt(D)

_NC = 2
_NS = 16
_NW = _NC * _NS
_FULL_BLOCKS = 7812
_CNT = 246


def _iota16():
    return lax.iota(jnp.int32, 16)


def _make_repack():
    mesh = plsc.VectorSubcoreMesh(core_axis_name="c", subcore_axis_name="s")

    @functools.partial(
        pl.kernel,
        out_type=jax.ShapeDtypeStruct((V // 4, 128), jnp.float32),
        mesh=mesh,
        scratch_types=[
            pltpu.VMEM((32, 128), jnp.float32),
            pltpu.VMEM((32, 128), jnp.float32),
            pltpu.VMEM((32, 128), jnp.float32),
            pltpu.VMEM((32, 128), jnp.float32),
            pltpu.SemaphoreType.DMA,
            pltpu.SemaphoreType.DMA,
            pltpu.SemaphoreType.DMA,
            pltpu.SemaphoreType.DMA,
        ],
        compiler_params=pltpu.CompilerParams(
            use_tc_tiling_on_sc=True, needs_layout_passes=False
        ),
    )
    def repack(tabT_hbm, tail_hbm, packed_hbm, s0, s1, p0, p1, si0, si1, so0, so1):
        wid = lax.axis_index("s") * _NC + lax.axis_index("c")
        base = wid * 244 + jnp.minimum(wid, 5)
        s_v = (s0, s1)
        p_v = (p0, p1)
        sem_i = (si0, si1)
        sem_o = (so0, so1)
        it16 = _iota16()
        rows01 = (it16, it16 + 16)

        def blkid(i):
            return jnp.minimum(base + i, _FULL_BLOCKS - 1)

        def in_copy(i, p):
            v0 = pl.multiple_of(blkid(i) * 128, 128)
            return pltpu.make_async_copy(
                tabT_hbm.at[:, pl.ds(v0, 128)], s_v[p], sem_i[p]
            )

        def out_copy(i, p):
            p0_ = pl.multiple_of(blkid(i) * 32, 32)
            return pltpu.make_async_copy(
                p_v[p], packed_hbm.at[pl.ds(p0_, 32)], sem_o[p]
            )

        in_copy(0, 0).start()
        in_copy(1, 1).start()

        def pair(itn, _):
            for p in (0, 1):
                i = itn * 2 + p
                in_copy(i, p).wait()

                @pl.when(itn >= 1)
                def _wait_out():
                    out_copy(i - 2, p).wait()

                for r in range(32):
                    for g in range(8):
                        cols = jnp.full((16,), 4 * r + g // 2, jnp.int32)
                        val = plsc.load_gather(s_v[p], [rows01[g % 2], cols])
                        p_v[p][r, pl.ds(16 * g, 16)] = val * SCALE
                out_copy(i, p).start()

                @pl.when(i + 2 < _CNT)
                def _prefetch():
                    in_copy(i + 2, p).start()

            return 0

        lax.fori_loop(0, _CNT // 2, pair, 0)
        out_copy(_CNT - 2, 0).wait()
        out_copy(_CNT - 1, 1).wait()

        @pl.when(wid == _NW - 1)
        def _tail():
            pltpu.sync_copy(tail_hbm, p0.at[pl.ds(0, 16)])
            pltpu.sync_copy(
                p0.at[pl.ds(0, 16)],
                packed_hbm.at[pl.ds(_FULL_BLOCKS * 32, 16)],
            )

    return repack


def _make_lookup():
    mesh = plsc.VectorSubcoreMesh(core_axis_name="c", subcore_axis_name="s")

    @functools.partial(
        pl.kernel,
        out_type=jax.ShapeDtypeStruct((200, D, 4096), jnp.float32),
        mesh=mesh,
        scratch_types=[
            pltpu.VMEM((200, 128), jnp.int32),
            pltpu.VMEM((128,), jnp.int32),
            pltpu.VMEM((128,), jnp.int32),
            pltpu.VMEM((128, 128), jnp.float32),
            pltpu.VMEM((128, 128), jnp.float32),
            pltpu.VMEM((32, 128), jnp.float32),
            pltpu.VMEM((32, 128), jnp.float32),
            pltpu.SemaphoreType.DMA,
            pltpu.SemaphoreType.DMA,
            pltpu.SemaphoreType.DMA,
            pltpu.SemaphoreType.DMA,
        ],
        compiler_params=pltpu.CompilerParams(
            use_tc_tiling_on_sc=True, needs_layout_passes=False
        ),
    )
    def lookup(
        xT_hbm, packed_hbm, out_hbm,
        xidx, jr0, jr1, g0, g1, o0, o1, sg0, sg1, soo0, soo1,
    ):
        wid = lax.axis_index("s") * _NC + lax.axis_index("c")
        lane0 = wid * 128
        jrs = (jr0, jr1)
        g_v = (g0, g1)
        o_v = (o0, o1)
        sem_g = (sg0, sg1)
        sem_o = (soo0, soo1)
        it16 = _iota16()
        rowsg = [it16 + 16 * g for g in range(8)]

        pltpu.sync_copy(xT_hbm.at[:, pl.ds(lane0, 128)], xidx)

        def prep(s, p):
            for g in range(8):
                v = xidx[s, pl.ds(16 * g, 16)]
                jrs[p][pl.ds(16 * g, 16)] = lax.shift_right_logical(v, 2)

        def gcopy(p):
            return pltpu.make_async_copy(
                packed_hbm.at[jrs[p]], g_v[p], sem_g[p]
            )

        def ocopy(s, p):
            return pltpu.make_async_copy(
                o_v[p], out_hbm.at[s, :, pl.ds(lane0, 128)], sem_o[p]
            )

        prep(0, 0)
        gcopy(0).start()
        prep(1, 1)
        gcopy(1).start()

        def pair(itn, _):
            for p in (0, 1):
                s = itn * 2 + p
                gcopy(p).wait()

                @pl.when(itn >= 1)
                def _wait_out():
                    ocopy(s - 2, p).wait()

                koffs = [
                    lax.shift_left(
                        jnp.bitwise_and(xidx[s, pl.ds(16 * g, 16)], 3), 5
                    )
                    for g in range(8)
                ]
                for dh in range(4):
                    for dl in range(8):
                        d = dh * 8 + dl
                        for g in range(8):
                            o_v[p][d, pl.ds(16 * g, 16)] = plsc.load_gather(
                                g_v[p], [rowsg[g], koffs[g] + d]
                            )
                ocopy(s, p).start()

                @pl.when(itn < 99)
                def _prefetch():
                    prep(s + 2, p)
                    gcopy(p).start()

            return 0

        lax.fori_loop(0, 100, pair, 0)
        ocopy(198, 0).wait()
        ocopy(199, 1).wait()

    return lookup


def kernel(x, table):
    xT = x.T
    tabT = table.T
    tail = (table[_FULL_BLOCKS * 128 :, :] * SCALE).reshape(16, 128)
    packed = _make_repack()(tabT, tail)
    outP = _make_lookup()(xT, packed)
    return outP.transpose(2, 0, 1)


# speedup vs baseline: 5.7528x; 4.6549x over previous
import functools
import math

import jax
import jax.numpy as jnp
from jax import lax
from jax.experimental import pallas as pl
from jax.experimental.pallas import tpu as pltpu
from jax.experimental.pallas import tpu_sc as plsc

D = 32
V = 1000000
SCALE = math.sqrt(D)

_NC = 2
_NS = 16
_NW = _NC * _NS
_FULL_BLOCKS = 7812
_CNT = 246


def _iota16():
    return lax.iota(jnp.int32, 16)


def _make_repack():
    mesh = plsc.VectorSubcoreMesh(core_axis_name="c", subcore_axis_name="s")

    @functools.partial(
        pl.kernel,
        out_type=jax.ShapeDtypeStruct((V // 4, 128), jnp.float32),
        mesh=mesh,
        scratch_types=[
            pltpu.VMEM((32, 128), jnp.float32),
            pltpu.VMEM((32, 128), jnp.float32),
            pltpu.VMEM((32, 128), jnp.float32),
            pltpu.VMEM((32, 128), jnp.float32),
            pltpu.SemaphoreType.DMA,
            pltpu.SemaphoreType.DMA,
            pltpu.SemaphoreType.DMA,
            pltpu.SemaphoreType.DMA,
        ],
        compiler_params=pltpu.CompilerParams(
            use_tc_tiling_on_sc=True, needs_layout_passes=False
        ),
    )
    def repack(tabT_hbm, tail_hbm, packed_hbm, s0, s1, p0, p1, si0, si1, so0, so1):
        wid = lax.axis_index("s") * _NC + lax.axis_index("c")
        base = wid * 244 + jnp.minimum(wid, 5)
        s_v = (s0, s1)
        p_v = (p0, p1)
        sem_i = (si0, si1)
        sem_o = (so0, so1)
        it16 = _iota16()
        rows01 = (it16, it16 + 16)

        def blkid(i):
            return jnp.minimum(base + i, _FULL_BLOCKS - 1)

        def in_copy(i, p):
            v0 = pl.multiple_of(blkid(i) * 128, 128)
            return pltpu.make_async_copy(
                tabT_hbm.at[:, pl.ds(v0, 128)], s_v[p], sem_i[p]
            )

        def out_copy(i, p):
            p0_ = pl.multiple_of(blkid(i) * 32, 32)
            return pltpu.make_async_copy(
                p_v[p], packed_hbm.at[pl.ds(p0_, 32)], sem_o[p]
            )

        in_copy(0, 0).start()
        in_copy(1, 1).start()

        def pair(itn, _):
            for p in (0, 1):
                i = itn * 2 + p
                in_copy(i, p).wait()

                @pl.when(itn >= 1)
                def _wait_out():
                    out_copy(i - 2, p).wait()

                @functools.partial(plsc.parallel_loop, 0, 32, unroll=4)
                def _row(r):
                    for g in range(8):
                        cols = jnp.full((16,), 0, jnp.int32) + (4 * r + g // 2)
                        val = plsc.load_gather(s_v[p], [rows01[g % 2], cols])
                        p_v[p][r, pl.ds(16 * g, 16)] = val * SCALE

                out_copy(i, p).start()

                @pl.when(i + 2 < _CNT)
                def _prefetch():
                    in_copy(i + 2, p).start()

            return 0

        lax.fori_loop(0, _CNT // 2, pair, 0)
        out_copy(_CNT - 2, 0).wait()
        out_copy(_CNT - 1, 1).wait()

        @pl.when(wid == _NW - 1)
        def _tail():
            pltpu.sync_copy(tail_hbm, p0.at[pl.ds(0, 16)])
            pltpu.sync_copy(
                p0.at[pl.ds(0, 16)],
                packed_hbm.at[pl.ds(_FULL_BLOCKS * 32, 16)],
            )

    return repack


def _make_lookup():
    mesh = plsc.VectorSubcoreMesh(core_axis_name="c", subcore_axis_name="s")

    @functools.partial(
        pl.kernel,
        out_type=jax.ShapeDtypeStruct((200, D, 4096), jnp.float32),
        mesh=mesh,
        scratch_types=[
            pltpu.VMEM((200, 128), jnp.int32),
            pltpu.VMEM((128,), jnp.int32),
            pltpu.VMEM((128,), jnp.int32),
            pltpu.VMEM((128, 128), jnp.float32),
            pltpu.VMEM((128, 128), jnp.float32),
            pltpu.VMEM((32, 128), jnp.float32),
            pltpu.VMEM((32, 128), jnp.float32),
            pltpu.SemaphoreType.DMA,
            pltpu.SemaphoreType.DMA,
            pltpu.SemaphoreType.DMA,
            pltpu.SemaphoreType.DMA,
        ],
        compiler_params=pltpu.CompilerParams(
            use_tc_tiling_on_sc=True, needs_layout_passes=False
        ),
    )
    def lookup(
        xT_hbm, packed_hbm, out_hbm,
        xidx, jr0, jr1, g0, g1, o0, o1, sg0, sg1, soo0, soo1,
    ):
        wid = lax.axis_index("s") * _NC + lax.axis_index("c")
        lane0 = wid * 128
        jrs = (jr0, jr1)
        g_v = (g0, g1)
        o_v = (o0, o1)
        sem_g = (sg0, sg1)
        sem_o = (soo0, soo1)
        it16 = _iota16()
        rowsg = [it16 + 16 * g for g in range(8)]

        pltpu.sync_copy(xT_hbm.at[:, pl.ds(lane0, 128)], xidx)

        def prep(s, p):
            for g in range(8):
                v = xidx[s, pl.ds(16 * g, 16)]
                jrs[p][pl.ds(16 * g, 16)] = lax.shift_right_logical(v, 2)

        def gcopy(p):
            return pltpu.make_async_copy(
                packed_hbm.at[jrs[p]], g_v[p], sem_g[p]
            )

        def ocopy(s, p):
            return pltpu.make_async_copy(
                o_v[p], out_hbm.at[s, :, pl.ds(lane0, 128)], sem_o[p]
            )

        prep(0, 0)
        gcopy(0).start()
        prep(1, 1)
        gcopy(1).start()

        def pair(itn, _):
            for p in (0, 1):
                s = itn * 2 + p
                gcopy(p).wait()

                @pl.when(itn >= 1)
                def _wait_out():
                    ocopy(s - 2, p).wait()

                koffs = [
                    lax.shift_left(
                        jnp.bitwise_and(xidx[s, pl.ds(16 * g, 16)], 3), 5
                    )
                    for g in range(8)
                ]
                @functools.partial(plsc.parallel_loop, 0, D, unroll=4)
                def _d(d):
                    for g in range(8):
                        o_v[p][d, pl.ds(16 * g, 16)] = plsc.load_gather(
                            g_v[p], [rowsg[g], koffs[g] + d]
                        )

                ocopy(s, p).start()

                @pl.when(itn < 99)
                def _prefetch():
                    prep(s + 2, p)
                    gcopy(p).start()

            return 0

        lax.fori_loop(0, 100, pair, 0)
        ocopy(198, 0).wait()
        ocopy(199, 1).wait()

    return lookup


def kernel(x, table):
    xT = x.T
    tabT = table.T
    tail = (table[_FULL_BLOCKS * 128 :, :] * SCALE).reshape(16, 128)
    packed = _make_repack()(tabT, tail)
    outP = _make_lookup()(xT, packed)
    return outP.transpose(2, 0, 1)
